# baseline (speedup 1.0000x reference)
import jax
import jax.numpy as jnp
from jax import lax
from jax.experimental import pallas as pl
from jax.experimental.pallas import tpu as pltpu

B, N, M = 16, 2048, 1024
NBLK = 8
NB = N // NBLK


def _body(x_ref, o_ref, minv, mini):
    nb = pl.program_id(1)
    vals = x_ref[0]
    lmin = jnp.min(vals, axis=0)
    iota = lax.broadcasted_iota(jnp.int32, vals.shape, 0)
    lidx = jnp.min(jnp.where(vals == lmin[None, :], iota, jnp.int32(N)), axis=0)
    lidx = lidx + nb * NB

    @pl.when(nb == 0)
    def _():
        minv[...] = lmin
        mini[...] = lidx

    @pl.when(nb > 0)
    def _():
        prev = minv[...]
        pred = lmin < prev
        minv[...] = jnp.where(pred, lmin, prev)
        mini[...] = jnp.where(pred, lidx, mini[...])

    @pl.when(nb == NBLK - 1)
    def _():
        o_ref[0, 0] = mini[...]


def kernel(x):
    out = pl.pallas_call(
        _body,
        grid=(B, NBLK),
        in_specs=[pl.BlockSpec((1, NB, M), lambda b, n: (b, n, 0))],
        out_specs=pl.BlockSpec((1, 1, M), lambda b, n: (b, 0, 0)),
        out_shape=jax.ShapeDtypeStruct((B, 1, M), jnp.int32),
        scratch_shapes=[
            pltpu.VMEM((M,), jnp.float32),
            pltpu.VMEM((M,), jnp.int32),
        ],
    )(x)
    return out.reshape(B, M)
